# baseline (speedup 1.0000x reference)
import jax
import jax.numpy as jnp
from jax import lax
from jax.experimental import pallas as pl
from jax.experimental.pallas import tpu as pltpu
from jax.experimental.pallas import tpu_sc as plsc

N_NODES = 10000
D = 128
E_SIGN = 320000
B_NODES = 8192

N_TILES = 16
HALF = 5120
ACC_ROWS = 5248
ACC_TILE = ACC_ROWS // N_TILES
CNT_ROWS = 6144
CNT_TILE = CNT_ROWS // N_TILES
DUMP = HALF
OUT_PAD = B_NODES + 128
E_PAD = 327680
E_TILE = E_PAD // N_TILES
CHUNK = 1024
SUB = CHUNK // 128
SLOTS = 5
ITERS = E_TILE // CHUNK
EROWS_TILE = E_TILE // 128
NODE_ROWS_TILE = B_NODES // N_TILES // 128
SELF_ROWS_TILE = 2


def _sc_body(x_hbm, psrc, pdst, nsrc, ndst, nodes_a, zacc, zcnt, ones_hbm,
             self_out, pos_out, pcnt_out, neg_out, ncnt_out,
             big_v, idx_s, idx_d, idx_dl, nodes_v, lid_blk, opos_blk,
             cntb_v, ones_v, ssum, scnt, sem_g, sem_s,
             g0, g1, g2, g3, g4, s0, s1, s2, s3, s4):
    c = lax.axis_index("c")
    s = lax.axis_index("s")
    gsem = (g0, g1, g2, g3, g4)
    ssem = (s0, s1, s2, s3, s4)

    pltpu.sync_copy(ones_hbm, ones_v)
    pltpu.sync_copy(nodes_a.at[s], nodes_v)

    selfs = [
        pltpu.async_copy(x_hbm.at[nodes_v.at[c * SELF_ROWS_TILE + j]],
                         big_v.at[pl.ds(j * 128, 128)], sem_g)
        for j in range(SELF_ROWS_TILE)
    ]
    for cp in selfs:
        cp.wait()
    nchunk = NODE_ROWS_TILE * 128
    schunk = SELF_ROWS_TILE * 128
    pltpu.sync_copy(big_v.at[pl.ds(0, schunk)],
                    self_out.at[pl.ds(s * nchunk + c * schunk, schunk)])

    def accumulate(src2, dst2, lo):
        def chunk(i, carry):
            row0 = s * EROWS_TILE + i * SUB
            pltpu.sync_copy(src2.at[pl.ds(row0, SUB)], idx_s)
            gathers = {
                r: pltpu.async_copy(x_hbm.at[idx_s.at[r]],
                                    big_v.at[pl.ds(r * 128, 128)], gsem[r])
                for r in range(SLOTS)
            }
            pltpu.sync_copy(dst2.at[pl.ds(row0, SUB)], idx_d)
            for r in range(SUB):
                for k in range(8):
                    v = idx_d[r, pl.ds(k * 16, 16)]
                    local = v - lo
                    ok = (local >= 0) & (local < HALF)
                    idx_dl[r, pl.ds(k * 16, 16)] = jnp.where(ok, local, DUMP)

            pending = {}
            for r in range(SUB):
                slot = r % SLOTS
                gathers[r].wait()
                a1 = pltpu.async_copy(big_v.at[pl.ds(slot * 128, 128)],
                                     ssum.at[idx_dl.at[r]], ssem[slot],
                                     add=True)
                a2 = pltpu.async_copy(ones_v, scnt.at[idx_dl.at[r]],
                                     ssem[slot], add=True)
                if r + SLOTS < SUB:
                    a1.wait()
                    a2.wait()
                    gathers[r + SLOTS] = pltpu.async_copy(
                        x_hbm.at[idx_s.at[r + SLOTS]],
                        big_v.at[pl.ds(slot * 128, 128)], gsem[slot])
                else:
                    pending[slot] = (a1, a2)
            for a1, a2 in pending.values():
                a1.wait()
                a2.wait()
            return carry
        lax.fori_loop(0, ITERS, chunk, 0)

    def emit(out_ref, cnt_ref, lo):
        lane = lax.iota(jnp.int32, 16)
        for j in range(NODE_ROWS_TILE):
            for k in range(8):
                v = nodes_v[j, pl.ds(k * 16, 16)]
                local = v - lo
                ok = (local >= 0) & (local < HALF)
                lid_blk[j, pl.ds(k * 16, 16)] = jnp.where(ok, local, DUMP)
                gpos = s * nchunk + j * 128 + k * 16 + lane
                opos_blk[j, pl.ds(k * 16, 16)] = jnp.where(ok, gpos, B_NODES)
        ins = [
            pltpu.async_copy(ssum.at[lid_blk.at[j]],
                             big_v.at[pl.ds(j * 128, 128)], sem_g)
            for j in range(NODE_ROWS_TILE)
        ] + [
            pltpu.async_copy(scnt.at[lid_blk.at[j]], cntb_v.at[j], sem_s)
            for j in range(NODE_ROWS_TILE)
        ]
        for cp in ins:
            cp.wait()
        outs = [
            pltpu.async_copy(big_v.at[pl.ds(j * 128, 128)],
                             out_ref.at[opos_blk.at[j]], sem_g)
            for j in range(NODE_ROWS_TILE)
        ] + [
            pltpu.async_copy(cntb_v.at[j], cnt_ref.at[opos_blk.at[j]], sem_s)
            for j in range(NODE_ROWS_TILE)
        ]
        for cp in outs:
            cp.wait()

    for h in range(2):
        lo = h * HALF
        pltpu.sync_copy(zacc, ssum.at[pl.ds(s * ACC_TILE, ACC_TILE)])
        pltpu.sync_copy(zcnt, scnt.at[pl.ds(s * CNT_TILE, CNT_TILE)])
        plsc.subcore_barrier()

        @pl.when(c == 0)
        def _():
            accumulate(psrc, pdst, lo)

        @pl.when(c == 1)
        def _():
            accumulate(nsrc, ndst, lo)

        plsc.subcore_barrier()

        @pl.when(c == 0)
        def _():
            emit(pos_out, pcnt_out, lo)

        @pl.when(c == 1)
        def _():
            emit(neg_out, ncnt_out, lo)

        plsc.subcore_barrier()


_sc_aggregate = pl.kernel(
    _sc_body,
    out_type=(
        jax.ShapeDtypeStruct((B_NODES, D), jnp.float32),
        jax.ShapeDtypeStruct((OUT_PAD, D), jnp.float32),
        jax.ShapeDtypeStruct((OUT_PAD,), jnp.float32),
        jax.ShapeDtypeStruct((OUT_PAD, D), jnp.float32),
        jax.ShapeDtypeStruct((OUT_PAD,), jnp.float32),
    ),
    mesh=plsc.VectorSubcoreMesh(core_axis_name="c", subcore_axis_name="s"),
    scratch_types=(
        pltpu.VMEM((SLOTS * 128, D), jnp.float32),
        pltpu.VMEM((SUB, 128), jnp.int32),
        pltpu.VMEM((SUB, 128), jnp.int32),
        pltpu.VMEM((SUB, 128), jnp.int32),
        pltpu.VMEM((NODE_ROWS_TILE, 128), jnp.int32),
        pltpu.VMEM((NODE_ROWS_TILE, 128), jnp.int32),
        pltpu.VMEM((NODE_ROWS_TILE, 128), jnp.int32),
        pltpu.VMEM((NODE_ROWS_TILE, 128), jnp.float32),
        pltpu.VMEM((128,), jnp.float32),
        pltpu.VMEM_SHARED((ACC_ROWS, D), jnp.float32),
        pltpu.VMEM_SHARED((CNT_ROWS,), jnp.float32),
        pltpu.SemaphoreType.DMA,
        pltpu.SemaphoreType.DMA,
        pltpu.SemaphoreType.DMA,
        pltpu.SemaphoreType.DMA,
        pltpu.SemaphoreType.DMA,
        pltpu.SemaphoreType.DMA,
        pltpu.SemaphoreType.DMA,
        pltpu.SemaphoreType.DMA,
        pltpu.SemaphoreType.DMA,
        pltpu.SemaphoreType.DMA,
        pltpu.SemaphoreType.DMA,
        pltpu.SemaphoreType.DMA,
    ),
)


def _tc_body(self_ref, pos_ref, pcnt_ref, neg_ref, ncnt_ref, w_ref, o_ref):
    pos_mean = pos_ref[...] / jnp.maximum(pcnt_ref[...], 1.0)
    neg_mean = neg_ref[...] / jnp.maximum(ncnt_ref[...], 1.0)
    h = jnp.dot(self_ref[...], w_ref[0:D, :], preferred_element_type=jnp.float32)
    h = h + jnp.dot(pos_mean, w_ref[D:2 * D, :], preferred_element_type=jnp.float32)
    h = h + jnp.dot(neg_mean, w_ref[2 * D:3 * D, :], preferred_element_type=jnp.float32)
    o_ref[...] = jnp.tanh(h)


_TC_BLOCK = 512
_tc_combine = pl.pallas_call(
    _tc_body,
    grid=(B_NODES // _TC_BLOCK,),
    in_specs=[
        pl.BlockSpec((_TC_BLOCK, D), lambda i: (i, 0)),
        pl.BlockSpec((_TC_BLOCK, D), lambda i: (i, 0)),
        pl.BlockSpec((_TC_BLOCK, 1), lambda i: (i, 0)),
        pl.BlockSpec((_TC_BLOCK, D), lambda i: (i, 0)),
        pl.BlockSpec((_TC_BLOCK, 1), lambda i: (i, 0)),
        pl.BlockSpec((3 * D, D), lambda i: (0, 0)),
    ],
    out_specs=pl.BlockSpec((_TC_BLOCK, D), lambda i: (i, 0)),
    out_shape=jax.ShapeDtypeStruct((B_NODES, D), jnp.float32),
)


def kernel(x, pos_edge_index, neg_edge_index, nodes, W):
    e = pos_edge_index.shape[1]
    pad = E_PAD - e
    pad_src = jnp.zeros((pad,), jnp.int32)
    pad_dst = jnp.full((pad,), 2 * HALF, jnp.int32)

    def prep(edge_index):
        src = jnp.concatenate([edge_index[0], pad_src]).reshape(E_PAD // 128, 128)
        dst = jnp.concatenate([edge_index[1], pad_dst]).reshape(E_PAD // 128, 128)
        return src, dst

    psrc, pdst = prep(pos_edge_index)
    nsrc, ndst = prep(neg_edge_index)
    nodes_a = nodes.reshape(N_TILES, NODE_ROWS_TILE, 128)
    zacc = jnp.zeros((ACC_TILE, D), jnp.float32)
    zcnt = jnp.zeros((CNT_TILE,), jnp.float32)
    ones = jnp.ones((128,), jnp.float32)

    self_f, pos_s, pos_c, neg_s, neg_c = _sc_aggregate(
        x, psrc, pdst, nsrc, ndst, nodes_a, zacc, zcnt, ones)
    return _tc_combine(self_f, pos_s, pos_c.reshape(OUT_PAD, 1),
                       neg_s, neg_c.reshape(OUT_PAD, 1), W)


# speedup vs baseline: 3.3998x; 3.3998x over previous
import jax
import jax.numpy as jnp
from jax import lax
from jax.experimental import pallas as pl
from jax.experimental.pallas import tpu as pltpu
from jax.experimental.pallas import tpu_sc as plsc

N_NODES = 10000
D = 128
B_NODES = 8192

N_TILES = 16
ACC_ROWS = 10112
ACC_TILE = ACC_ROWS // N_TILES
CNT_ROWS = 10240
CNT_TILE = CNT_ROWS // N_TILES
DUMP = N_NODES
E_PAD = 327680
E_TILE = E_PAD // N_TILES
CHUNK = 1024
SUB = CHUNK // 128
SLOTS = 2
ITERS = E_TILE // CHUNK
EROWS_TILE = E_TILE // 128
NODE_ROWS_TILE = B_NODES // N_TILES // 128
SELF_ROWS_TILE = 2


def _sc_body(x_hbm, psrc, pdst, nsrc, ndst, nodes_a, zacc, zcnt, ones_hbm,
             self_out, pos_out, pcnt_out, neg_out, ncnt_out,
             big_v, idx_s, idx_d, nodes_v, cntb_v, ones_v, ssum, scnt,
             g0, g1, s0, s1):
    c = lax.axis_index("c")
    s = lax.axis_index("s")
    gsem = (g0, g1)
    ssem = (s0, s1)

    pltpu.sync_copy(ones_hbm, ones_v)
    pltpu.sync_copy(nodes_a.at[s], nodes_v)

    pltpu.sync_copy(zacc, ssum.at[pl.ds(s * ACC_TILE, ACC_TILE)])
    pltpu.sync_copy(zcnt, scnt.at[pl.ds(s * CNT_TILE, CNT_TILE)])

    for j in range(SELF_ROWS_TILE):
        slot = j % SLOTS
        cp = pltpu.async_copy(x_hbm.at[nodes_v.at[c * SELF_ROWS_TILE + j]],
                              big_v.at[pl.ds(slot * 128, 128)], gsem[slot])
        cp.wait()
        pltpu.sync_copy(
            big_v.at[pl.ds(slot * 128, 128)],
            self_out.at[pl.ds(s * 512 + c * (SELF_ROWS_TILE * 128) + j * 128,
                              128)])

    plsc.subcore_barrier()

    def accumulate(src2, dst2):
        def chunk(i, carry):
            row0 = s * EROWS_TILE + i * SUB
            pltpu.sync_copy(src2.at[pl.ds(row0, SUB)], idx_s)
            gathers = {
                r: pltpu.async_copy(x_hbm.at[idx_s.at[r]],
                                    big_v.at[pl.ds(r * 128, 128)], gsem[r])
                for r in range(SLOTS)
            }
            pltpu.sync_copy(dst2.at[pl.ds(row0, SUB)], idx_d)

            pending = {}
            for r in range(SUB):
                slot = r % SLOTS
                gathers[r].wait()
                a1 = pltpu.async_copy(big_v.at[pl.ds(slot * 128, 128)],
                                      ssum.at[idx_d.at[r]], ssem[slot],
                                      add=True)
                a2 = pltpu.async_copy(ones_v, scnt.at[idx_d.at[r]],
                                      ssem[slot], add=True)
                if r + SLOTS < SUB:
                    a1.wait()
                    a2.wait()
                    gathers[r + SLOTS] = pltpu.async_copy(
                        x_hbm.at[idx_s.at[r + SLOTS]],
                        big_v.at[pl.ds(slot * 128, 128)], gsem[slot])
                else:
                    pending[slot] = (a1, a2)
            for a1, a2 in pending.values():
                a1.wait()
                a2.wait()
            return carry
        lax.fori_loop(0, ITERS, chunk, 0)

    @pl.when(c == 0)
    def _():
        accumulate(psrc, pdst)

    @pl.when(c == 1)
    def _():
        accumulate(nsrc, ndst)

    plsc.subcore_barrier()

    def emit(out_ref, cnt_ref):
        for j in range(NODE_ROWS_TILE):
            slot = j % SLOTS
            cp = pltpu.async_copy(ssum.at[nodes_v.at[j]],
                                  big_v.at[pl.ds(slot * 128, 128)], gsem[slot])
            cc = pltpu.async_copy(scnt.at[nodes_v.at[j]], cntb_v.at[j],
                                  ssem[slot])
            cp.wait()
            cc.wait()
            pltpu.sync_copy(big_v.at[pl.ds(slot * 128, 128)],
                            out_ref.at[pl.ds(s * 512 + j * 128, 128)])
            pltpu.sync_copy(cntb_v.at[j],
                            cnt_ref.at[pl.ds(s * 512 + j * 128, 128)])

    @pl.when(c == 0)
    def _():
        emit(pos_out, pcnt_out)

    @pl.when(c == 1)
    def _():
        emit(neg_out, ncnt_out)


_sc_aggregate = pl.kernel(
    _sc_body,
    out_type=(
        jax.ShapeDtypeStruct((B_NODES, D), jnp.float32),
        jax.ShapeDtypeStruct((B_NODES, D), jnp.float32),
        jax.ShapeDtypeStruct((B_NODES,), jnp.float32),
        jax.ShapeDtypeStruct((B_NODES, D), jnp.float32),
        jax.ShapeDtypeStruct((B_NODES,), jnp.float32),
    ),
    mesh=plsc.VectorSubcoreMesh(core_axis_name="c", subcore_axis_name="s"),
    scratch_types=(
        pltpu.VMEM((SLOTS * 128, D), jnp.float32),
        pltpu.VMEM((SUB, 128), jnp.int32),
        pltpu.VMEM((SUB, 128), jnp.int32),
        pltpu.VMEM((NODE_ROWS_TILE, 128), jnp.int32),
        pltpu.VMEM((NODE_ROWS_TILE, 128), jnp.float32),
        pltpu.VMEM((128,), jnp.float32),
        pltpu.VMEM_SHARED((ACC_ROWS, D), jnp.float32),
        pltpu.VMEM_SHARED((CNT_ROWS,), jnp.float32),
        pltpu.SemaphoreType.DMA,
        pltpu.SemaphoreType.DMA,
        pltpu.SemaphoreType.DMA,
        pltpu.SemaphoreType.DMA,
    ),
)


def _tc_body(self_ref, pos_ref, pcnt_ref, neg_ref, ncnt_ref, w_ref, o_ref):
    pos_mean = pos_ref[...] / jnp.maximum(pcnt_ref[...], 1.0)
    neg_mean = neg_ref[...] / jnp.maximum(ncnt_ref[...], 1.0)
    h = jnp.dot(self_ref[...], w_ref[0:D, :], preferred_element_type=jnp.float32)
    h = h + jnp.dot(pos_mean, w_ref[D:2 * D, :], preferred_element_type=jnp.float32)
    h = h + jnp.dot(neg_mean, w_ref[2 * D:3 * D, :], preferred_element_type=jnp.float32)
    o_ref[...] = jnp.tanh(h)


_TC_BLOCK = 512
_tc_combine = pl.pallas_call(
    _tc_body,
    grid=(B_NODES // _TC_BLOCK,),
    in_specs=[
        pl.BlockSpec((_TC_BLOCK, D), lambda i: (i, 0)),
        pl.BlockSpec((_TC_BLOCK, D), lambda i: (i, 0)),
        pl.BlockSpec((_TC_BLOCK, 1), lambda i: (i, 0)),
        pl.BlockSpec((_TC_BLOCK, D), lambda i: (i, 0)),
        pl.BlockSpec((_TC_BLOCK, 1), lambda i: (i, 0)),
        pl.BlockSpec((3 * D, D), lambda i: (0, 0)),
    ],
    out_specs=pl.BlockSpec((_TC_BLOCK, D), lambda i: (i, 0)),
    out_shape=jax.ShapeDtypeStruct((B_NODES, D), jnp.float32),
)


def kernel(x, pos_edge_index, neg_edge_index, nodes, W):
    e = pos_edge_index.shape[1]
    pad = E_PAD - e
    pad_src = jnp.zeros((pad,), jnp.int32)
    pad_dst = jnp.full((pad,), DUMP, jnp.int32)

    def prep(edge_index):
        src = jnp.concatenate([edge_index[0], pad_src]).reshape(E_PAD // 128, 128)
        dst = jnp.concatenate([edge_index[1], pad_dst]).reshape(E_PAD // 128, 128)
        return src, dst

    psrc, pdst = prep(pos_edge_index)
    nsrc, ndst = prep(neg_edge_index)
    nodes_a = nodes.reshape(N_TILES, NODE_ROWS_TILE, 128)
    zacc = jnp.zeros((ACC_TILE, D), jnp.float32)
    zcnt = jnp.zeros((CNT_TILE,), jnp.float32)
    ones = jnp.ones((128,), jnp.float32)

    self_f, pos_s, pos_c, neg_s, neg_c = _sc_aggregate(
        x, psrc, pdst, nsrc, ndst, nodes_a, zacc, zcnt, ones)
    return _tc_combine(self_f, pos_s, pos_c.reshape(B_NODES, 1),
                       neg_s, neg_c.reshape(B_NODES, 1), W)


# speedup vs baseline: 3.4596x; 1.0176x over previous
import jax
import jax.numpy as jnp
from jax import lax
from jax.experimental import pallas as pl
from jax.experimental.pallas import tpu as pltpu
from jax.experimental.pallas import tpu_sc as plsc

N_NODES = 10000
D = 128
B_NODES = 8192

N_TILES = 16
ACC_ROWS = 10112
ACC_TILE = ACC_ROWS // N_TILES
CNT_ROWS = 10240
CNT_TILE = CNT_ROWS // N_TILES
DUMP = N_NODES
E_PAD = 327680
E_TILE = E_PAD // N_TILES
CHUNK = 5120
SUB = CHUNK // 128
SLOTS = 2
ITERS = E_TILE // CHUNK
EROWS_TILE = E_TILE // 128
NODE_ROWS_TILE = B_NODES // N_TILES // 128
SELF_ROWS_TILE = 2


def _sc_body(x_hbm, psrc, pdst, nsrc, ndst, nodes_a, zacc, zcnt, ones_hbm,
             self_out, pos_out, pcnt_out, neg_out, ncnt_out,
             big_v, idx_s, idx_d, nodes_v, cntb_v, ones_v, ssum, scnt,
             g0, g1, s0, s1):
    c = lax.axis_index("c")
    s = lax.axis_index("s")
    gsem = (g0, g1)
    ssem = (s0, s1)

    pltpu.sync_copy(ones_hbm, ones_v)
    pltpu.sync_copy(nodes_a.at[s], nodes_v)

    pltpu.sync_copy(zacc, ssum.at[pl.ds(s * ACC_TILE, ACC_TILE)])
    pltpu.sync_copy(zcnt, scnt.at[pl.ds(s * CNT_TILE, CNT_TILE)])

    for j in range(SELF_ROWS_TILE):
        slot = j % SLOTS
        cp = pltpu.async_copy(x_hbm.at[nodes_v.at[c * SELF_ROWS_TILE + j]],
                              big_v.at[pl.ds(slot * 128, 128)], gsem[slot])
        cp.wait()
        pltpu.sync_copy(
            big_v.at[pl.ds(slot * 128, 128)],
            self_out.at[pl.ds(s * 512 + c * (SELF_ROWS_TILE * 128) + j * 128,
                              128)])

    plsc.subcore_barrier()

    def accumulate(src2, dst2):
        def chunk(i, carry):
            row0 = s * EROWS_TILE + i * SUB
            pltpu.sync_copy(src2.at[pl.ds(row0, SUB)], idx_s)
            gathers = {
                r: pltpu.async_copy(x_hbm.at[idx_s.at[r]],
                                    big_v.at[pl.ds(r * 128, 128)], gsem[r])
                for r in range(SLOTS)
            }
            pltpu.sync_copy(dst2.at[pl.ds(row0, SUB)], idx_d)

            pending = {}
            for r in range(SUB):
                slot = r % SLOTS
                gathers[r].wait()
                a1 = pltpu.async_copy(big_v.at[pl.ds(slot * 128, 128)],
                                      ssum.at[idx_d.at[r]], ssem[slot],
                                      add=True)
                a2 = pltpu.async_copy(ones_v, scnt.at[idx_d.at[r]],
                                      ssem[slot], add=True)
                if r + SLOTS < SUB:
                    a1.wait()
                    a2.wait()
                    gathers[r + SLOTS] = pltpu.async_copy(
                        x_hbm.at[idx_s.at[r + SLOTS]],
                        big_v.at[pl.ds(slot * 128, 128)], gsem[slot])
                else:
                    pending[slot] = (a1, a2)
            for a1, a2 in pending.values():
                a1.wait()
                a2.wait()
            return carry
        lax.fori_loop(0, ITERS, chunk, 0)

    @pl.when(c == 0)
    def _():
        accumulate(psrc, pdst)

    @pl.when(c == 1)
    def _():
        accumulate(nsrc, ndst)

    plsc.subcore_barrier()

    def emit(out_ref, cnt_ref):
        for j in range(NODE_ROWS_TILE):
            slot = j % SLOTS
            cp = pltpu.async_copy(ssum.at[nodes_v.at[j]],
                                  big_v.at[pl.ds(slot * 128, 128)], gsem[slot])
            cc = pltpu.async_copy(scnt.at[nodes_v.at[j]], cntb_v.at[j],
                                  ssem[slot])
            cp.wait()
            cc.wait()
            pltpu.sync_copy(big_v.at[pl.ds(slot * 128, 128)],
                            out_ref.at[pl.ds(s * 512 + j * 128, 128)])
            pltpu.sync_copy(cntb_v.at[j],
                            cnt_ref.at[pl.ds(s * 512 + j * 128, 128)])

    @pl.when(c == 0)
    def _():
        emit(pos_out, pcnt_out)

    @pl.when(c == 1)
    def _():
        emit(neg_out, ncnt_out)


_sc_aggregate = pl.kernel(
    _sc_body,
    out_type=(
        jax.ShapeDtypeStruct((B_NODES, D), jnp.float32),
        jax.ShapeDtypeStruct((B_NODES, D), jnp.float32),
        jax.ShapeDtypeStruct((B_NODES,), jnp.float32),
        jax.ShapeDtypeStruct((B_NODES, D), jnp.float32),
        jax.ShapeDtypeStruct((B_NODES,), jnp.float32),
    ),
    mesh=plsc.VectorSubcoreMesh(core_axis_name="c", subcore_axis_name="s"),
    scratch_types=(
        pltpu.VMEM((SLOTS * 128, D), jnp.float32),
        pltpu.VMEM((SUB, 128), jnp.int32),
        pltpu.VMEM((SUB, 128), jnp.int32),
        pltpu.VMEM((NODE_ROWS_TILE, 128), jnp.int32),
        pltpu.VMEM((NODE_ROWS_TILE, 128), jnp.float32),
        pltpu.VMEM((128,), jnp.float32),
        pltpu.VMEM_SHARED((ACC_ROWS, D), jnp.float32),
        pltpu.VMEM_SHARED((CNT_ROWS,), jnp.float32),
        pltpu.SemaphoreType.DMA,
        pltpu.SemaphoreType.DMA,
        pltpu.SemaphoreType.DMA,
        pltpu.SemaphoreType.DMA,
    ),
)


def _tc_body(self_ref, pos_ref, pcnt_ref, neg_ref, ncnt_ref, w_ref, o_ref):
    pos_mean = pos_ref[...] / jnp.maximum(pcnt_ref[...], 1.0)
    neg_mean = neg_ref[...] / jnp.maximum(ncnt_ref[...], 1.0)
    h = jnp.dot(self_ref[...], w_ref[0:D, :], preferred_element_type=jnp.float32)
    h = h + jnp.dot(pos_mean, w_ref[D:2 * D, :], preferred_element_type=jnp.float32)
    h = h + jnp.dot(neg_mean, w_ref[2 * D:3 * D, :], preferred_element_type=jnp.float32)
    o_ref[...] = jnp.tanh(h)


_TC_BLOCK = 512
_tc_combine = pl.pallas_call(
    _tc_body,
    grid=(B_NODES // _TC_BLOCK,),
    in_specs=[
        pl.BlockSpec((_TC_BLOCK, D), lambda i: (i, 0)),
        pl.BlockSpec((_TC_BLOCK, D), lambda i: (i, 0)),
        pl.BlockSpec((_TC_BLOCK, 1), lambda i: (i, 0)),
        pl.BlockSpec((_TC_BLOCK, D), lambda i: (i, 0)),
        pl.BlockSpec((_TC_BLOCK, 1), lambda i: (i, 0)),
        pl.BlockSpec((3 * D, D), lambda i: (0, 0)),
    ],
    out_specs=pl.BlockSpec((_TC_BLOCK, D), lambda i: (i, 0)),
    out_shape=jax.ShapeDtypeStruct((B_NODES, D), jnp.float32),
)


def kernel(x, pos_edge_index, neg_edge_index, nodes, W):
    e = pos_edge_index.shape[1]
    pad = E_PAD - e
    pad_src = jnp.zeros((pad,), jnp.int32)
    pad_dst = jnp.full((pad,), DUMP, jnp.int32)

    def prep(edge_index):
        src = jnp.concatenate([edge_index[0], pad_src]).reshape(E_PAD // 128, 128)
        dst = jnp.concatenate([edge_index[1], pad_dst]).reshape(E_PAD // 128, 128)
        return src, dst

    psrc, pdst = prep(pos_edge_index)
    nsrc, ndst = prep(neg_edge_index)
    nodes_a = nodes.reshape(N_TILES, NODE_ROWS_TILE, 128)
    zacc = jnp.zeros((ACC_TILE, D), jnp.float32)
    zcnt = jnp.zeros((CNT_TILE,), jnp.float32)
    ones = jnp.ones((128,), jnp.float32)

    self_f, pos_s, pos_c, neg_s, neg_c = _sc_aggregate(
        x, psrc, pdst, nsrc, ndst, nodes_a, zacc, zcnt, ones)
    return _tc_combine(self_f, pos_s, pos_c.reshape(B_NODES, 1),
                       neg_s, neg_c.reshape(B_NODES, 1), W)
